# Initial kernel scaffold; baseline (speedup 1.0000x reference)
import functools

import jax
import jax.numpy as jnp
from jax import lax
from jax.experimental import pallas as pl
from jax.experimental.pallas import tpu as pltpu
from jax.experimental.pallas import tpu_sc as plsc

_N = 10000
_NP = 10240
_E = 320000
_W = 128
_EP = 327680
_R = _EP // _W
_NW = 32
_RPW = _R // _NW
_CH = 10
_NCH = _RPW // _CH

_BLK_E = 2048
_GRID_E = _EP // _BLK_E
_BLK_N = 1024
_GRID_N = _NP // _BLK_N

_F32 = jnp.float32


def _sp(x):
    return jnp.maximum(x, 0.0) + jnp.log1p(jnp.exp(-jnp.abs(x)))


def _dot(a, w):
    return jax.lax.dot_general(a, w, (((1,), (0,)), ((), ())),
                               preferred_element_type=_F32)


def _wspec(shape):
    return pl.BlockSpec(shape, lambda i: (0, 0))


def _rowspec(blk, width):
    return pl.BlockSpec((blk, width), lambda i: (i, 0))


def _embed_nodes(x_p, fa):
    (w1, b1), (w2, b2) = fa

    def body(x_ref, w1_ref, b1_ref, w2_ref, b2_ref, o_ref):
        t = _sp(_dot(x_ref[...], w1_ref[...]) + b1_ref[...])
        o_ref[...] = _dot(t, w2_ref[...]) + b2_ref[...]

    return pl.pallas_call(
        body,
        grid=(_GRID_N,),
        in_specs=[_rowspec(_BLK_N, 128), _wspec((128, 64)), _wspec((1, 64)),
                  _wspec((64, 64)), _wspec((1, 64))],
        out_specs=_rowspec(_BLK_N, 64),
        out_shape=jax.ShapeDtypeStruct((_NP, 64), _F32),
    )(x_p, w1, b1.reshape(1, 64), w2, b2.reshape(1, 64))


def _embed_edges(ea_p, fb):
    (w1, b1), (w2, b2) = fb

    def body(a_ref, w1_ref, b1_ref, w2_ref, b2_ref, o_ref):
        t = _sp(_dot(a_ref[...], w1_ref[...]) + b1_ref[...])
        o_ref[...] = _dot(t, w2_ref[...]) + b2_ref[...]

    return pl.pallas_call(
        body,
        grid=(_GRID_E,),
        in_specs=[_rowspec(_BLK_E, 16), _wspec((16, 64)), _wspec((1, 64)),
                  _wspec((64, 64)), _wspec((1, 64))],
        out_specs=_rowspec(_BLK_E, 64),
        out_shape=jax.ShapeDtypeStruct((_EP, 64), _F32),
    )(ea_p, w1, b1.reshape(1, 64), w2, b2.reshape(1, 64))


def _edge_step1(hs, hr, e0, fe, fv):
    (we1, be1), (we2, be2), (we3, be3) = fe
    (wv1, bv1), (wv2, bv2), (wv3, bv3) = fv
    wva, wvb = wv1[:64], wv1[64:]

    def body(hs_ref, hr_ref, e0_ref,
             we1_r, be1_r, we2_r, be2_r, we3_r, be3_r,
             wva_r, wvb_r, bv1_r, wv2_r, bv2_r, wv3_r, bv3_r,
             e1_o, upd_o):
        prod = hs_ref[...] * hr_ref[...]
        t = _sp(_dot(prod, we1_r[...]) + be1_r[...])
        t = _sp(_dot(t, we2_r[...]) + be2_r[...])
        e1 = _dot(t, we3_r[...]) + be3_r[...] + e0_ref[...]
        e1_o[...] = e1
        u = _sp(_dot(hr_ref[...], wva_r[...]) + _dot(e1, wvb_r[...]) + bv1_r[...])
        u = _sp(_dot(u, wv2_r[...]) + bv2_r[...])
        upd_o[...] = _dot(u, wv3_r[...]) + bv3_r[...]

    return pl.pallas_call(
        body,
        grid=(_GRID_E,),
        in_specs=[_rowspec(_BLK_E, 64)] * 3 +
                 [_wspec((64, 64)), _wspec((1, 64))] * 3 +
                 [_wspec((64, 64)), _wspec((64, 64)), _wspec((1, 64)),
                  _wspec((64, 64)), _wspec((1, 64)),
                  _wspec((64, 64)), _wspec((1, 64))],
        out_specs=[_rowspec(_BLK_E, 64)] * 2,
        out_shape=[jax.ShapeDtypeStruct((_EP, 64), _F32)] * 2,
    )(hs, hr, e0,
      we1, be1.reshape(1, 64), we2, be2.reshape(1, 64), we3, be3.reshape(1, 64),
      wva, wvb, bv1.reshape(1, 64), wv2, bv2.reshape(1, 64), wv3, bv3.reshape(1, 64))


def _edge_step2_force(hs, hr, e1, fe, mlp1):
    (we1, be1), (we2, be2), (we3, be3) = fe
    (wm1, bm1), (wm2, bm2), (wm3, bm3) = mlp1
    wm3p = jnp.pad(wm3, ((0, 0), (0, 13)))
    bm3p = jnp.pad(bm3.reshape(1, 3), ((0, 0), (0, 13)))

    def body(hs_ref, hr_ref, e1_ref,
             we1_r, be1_r, we2_r, be2_r, we3_r, be3_r,
             wm1_r, bm1_r, wm2_r, bm2_r, wm3_r, bm3_r,
             fij_o):
        prod = hs_ref[...] * hr_ref[...]
        t = _sp(_dot(prod, we1_r[...]) + be1_r[...])
        t = _sp(_dot(t, we2_r[...]) + be2_r[...])
        e2 = _dot(t, we3_r[...]) + be3_r[...] + e1_ref[...]
        f = _sp(_dot(e2, wm1_r[...]) + bm1_r[...])
        f = _sp(_dot(f, wm2_r[...]) + bm2_r[...])
        fij_o[...] = _dot(f, wm3_r[...]) + bm3_r[...]

    return pl.pallas_call(
        body,
        grid=(_GRID_E,),
        in_specs=[_rowspec(_BLK_E, 64)] * 3 +
                 [_wspec((64, 64)), _wspec((1, 64))] * 2 +
                 [_wspec((64, 64)), _wspec((1, 64))] * 2 +
                 [_wspec((64, 16)), _wspec((1, 16))],
        out_specs=_rowspec(_BLK_E, 16),
        out_shape=jax.ShapeDtypeStruct((_EP, 16), _F32),
    )(hs, hr, e1,
      we1, be1.reshape(1, 64), we2, be2.reshape(1, 64), we3, be3.reshape(1, 64),
      wm1, bm1.reshape(1, 64), wm2, bm2.reshape(1, 64), wm3p, bm3p)


def _combine_h(h0, p0, p1):
    def body(a_ref, b_ref, c_ref, o_ref):
        o_ref[...] = a_ref[...] + b_ref[...] + c_ref[...]

    return pl.pallas_call(
        body,
        grid=(_GRID_N,),
        in_specs=[_rowspec(_BLK_N, 64)] * 3,
        out_specs=_rowspec(_BLK_N, 64),
        out_shape=jax.ShapeDtypeStruct((_NP, 64), _F32),
    )(h0, p0, p1)


def _final_node(ar0, ar1, as0, as1, nt_p, mlp2):
    (w1, b1), (w2, b2), (w3, b3) = mlp2
    w1p = jnp.pad(w1, ((0, 3), (0, 3)))
    b1p = jnp.pad(b1.reshape(1, 5), ((0, 0), (0, 3)))
    w2p = jnp.pad(w2, ((0, 3), (0, 3)))
    b2p = jnp.pad(b2.reshape(1, 5), ((0, 0), (0, 3)))
    w3p = jnp.pad(w3, ((0, 3), (0, 7)))
    b3p = jnp.pad(b3.reshape(1, 1), ((0, 0), (0, 7)))

    def body(ar0_r, ar1_r, as0_r, as1_r, nt_r,
             w1_r, b1_r, w2_r, b2_r, w3_r, b3_r,
             ai_o, g_o):
        ai_o[...] = (ar0_r[...] + ar1_r[...]) - (as0_r[...] + as1_r[...])
        g = _sp(_dot(nt_r[...], w1_r[...]) + b1_r[...])
        g = _sp(_dot(g, w2_r[...]) + b2_r[...])
        g_o[...] = _sp(_dot(g, w3_r[...]) + b3_r[...])

    return pl.pallas_call(
        body,
        grid=(_GRID_N,),
        in_specs=[_rowspec(_BLK_N, 16)] * 4 + [_rowspec(_BLK_N, 8)] +
                 [_wspec((8, 8)), _wspec((1, 8))] * 3,
        out_specs=[_rowspec(_BLK_N, 16), _rowspec(_BLK_N, 8)],
        out_shape=[jax.ShapeDtypeStruct((_NP, 16), _F32),
                   jax.ShapeDtypeStruct((_NP, 8), _F32)],
    )(ar0, ar1, as0, as1, nt_p, w1p, b1p, w2p, b2p, w3p, b3p)


_MESH = plsc.VectorSubcoreMesh(core_axis_name="c", subcore_axis_name="s")


def _gather_two(h_pad, sidx, ridx):
    @functools.partial(
        pl.kernel,
        out_type=[jax.ShapeDtypeStruct((_R, _W, 64), _F32)] * 2,
        mesh=_MESH,
        scratch_types=[pltpu.VMEM((_CH, _W), jnp.int32),
                       pltpu.VMEM((_CH, _W, 64), _F32),
                       pltpu.SemaphoreType.DMA],
    )
    def k(h_hbm, s_hbm, r_hbm, hs_out, hr_out, idx_v, rows_v, sem):
        wid = lax.axis_index("s") * 2 + lax.axis_index("c")
        base = wid * _RPW

        def run(idx_hbm, out_hbm):
            def chunk(ci, carry):
                row0 = base + ci * _CH
                pltpu.sync_copy(idx_hbm.at[pl.ds(row0, _CH)], idx_v)
                cps = [pltpu.async_copy(h_hbm.at[idx_v.at[j]], rows_v.at[j], sem)
                       for j in range(_CH)]
                for cp in cps:
                    cp.wait()
                pltpu.sync_copy(rows_v, out_hbm.at[pl.ds(row0, _CH)])
                return carry
            lax.fori_loop(0, _NCH, chunk, 0)

        run(s_hbm, hs_out)
        run(r_hbm, hr_out)

    hs, hr = k(h_pad, sidx, ridx)
    return hs.reshape(_EP, 64), hr.reshape(_EP, 64)


def _scatter_add64(vals, ridx, zeros_n):
    @functools.partial(
        pl.kernel,
        out_type=[jax.ShapeDtypeStruct((_NP, 64), _F32)] * 2,
        mesh=_MESH,
        scratch_types=[pltpu.VMEM((_CH, _W), jnp.int32),
                       pltpu.VMEM((_CH, _W, 64), _F32),
                       pltpu.VMEM_SHARED((_NP, 64), _F32),
                       pltpu.SemaphoreType.DMA],
    )
    def k(v_hbm, i_hbm, z_hbm, p0_out, p1_out, idx_v, rows_v, acc, sem):
        cid = lax.axis_index("c")
        sid = lax.axis_index("s")
        nsl = _NP // 16
        pltpu.sync_copy(z_hbm.at[pl.ds(sid * nsl, nsl)],
                        acc.at[pl.ds(sid * nsl, nsl)])
        plsc.subcore_barrier()
        base = cid * (_R // 2) + sid * _RPW

        def chunk(ci, carry):
            row0 = base + ci * _CH
            pltpu.sync_copy(i_hbm.at[pl.ds(row0, _CH)], idx_v)
            pltpu.sync_copy(v_hbm.at[pl.ds(row0, _CH)], rows_v)
            for j in range(_CH):
                pltpu.sync_copy(rows_v.at[j], acc.at[idx_v.at[j]], add=True)
            return carry

        lax.fori_loop(0, _NCH, chunk, 0)
        plsc.subcore_barrier()

        @pl.when(cid == 0)
        def _():
            pltpu.sync_copy(acc.at[pl.ds(sid * nsl, nsl)],
                            p0_out.at[pl.ds(sid * nsl, nsl)])

        @pl.when(cid == 1)
        def _():
            pltpu.sync_copy(acc.at[pl.ds(sid * nsl, nsl)],
                            p1_out.at[pl.ds(sid * nsl, nsl)])

    return k(vals, ridx, zeros_n)


def _scatter_add16_two(vals, ridx, sidx, zeros_n16):
    @functools.partial(
        pl.kernel,
        out_type=[jax.ShapeDtypeStruct((_NP, 16), _F32)] * 4,
        mesh=_MESH,
        scratch_types=[pltpu.VMEM((_CH, _W), jnp.int32),
                       pltpu.VMEM((_CH, _W), jnp.int32),
                       pltpu.VMEM((_CH, _W, 16), _F32),
                       pltpu.VMEM_SHARED((_NP, 16), _F32),
                       pltpu.VMEM_SHARED((_NP, 16), _F32),
                       pltpu.SemaphoreType.DMA],
    )
    def k(v_hbm, ri_hbm, si_hbm, z_hbm,
          ar0_out, ar1_out, as0_out, as1_out,
          ridx_v, sidx_v, rows_v, accr, accs, sem):
        cid = lax.axis_index("c")
        sid = lax.axis_index("s")
        nsl = _NP // 16
        pltpu.sync_copy(z_hbm.at[pl.ds(sid * nsl, nsl)],
                        accr.at[pl.ds(sid * nsl, nsl)])
        pltpu.sync_copy(z_hbm.at[pl.ds(sid * nsl, nsl)],
                        accs.at[pl.ds(sid * nsl, nsl)])
        plsc.subcore_barrier()
        base = cid * (_R // 2) + sid * _RPW

        def chunk(ci, carry):
            row0 = base + ci * _CH
            pltpu.sync_copy(ri_hbm.at[pl.ds(row0, _CH)], ridx_v)
            pltpu.sync_copy(si_hbm.at[pl.ds(row0, _CH)], sidx_v)
            pltpu.sync_copy(v_hbm.at[pl.ds(row0, _CH)], rows_v)
            for j in range(_CH):
                pltpu.sync_copy(rows_v.at[j], accr.at[ridx_v.at[j]], add=True)
                pltpu.sync_copy(rows_v.at[j], accs.at[sidx_v.at[j]], add=True)
            return carry

        lax.fori_loop(0, _NCH, chunk, 0)
        plsc.subcore_barrier()

        @pl.when(cid == 0)
        def _():
            pltpu.sync_copy(accr.at[pl.ds(sid * nsl, nsl)],
                            ar0_out.at[pl.ds(sid * nsl, nsl)])
            pltpu.sync_copy(accs.at[pl.ds(sid * nsl, nsl)],
                            as0_out.at[pl.ds(sid * nsl, nsl)])

        @pl.when(cid == 1)
        def _():
            pltpu.sync_copy(accr.at[pl.ds(sid * nsl, nsl)],
                            ar1_out.at[pl.ds(sid * nsl, nsl)])
            pltpu.sync_copy(accs.at[pl.ds(sid * nsl, nsl)],
                            as1_out.at[pl.ds(sid * nsl, nsl)])

    return k(vals, ridx, sidx, zeros_n16)


def kernel(x, edge_attr, node_type, edge_index, fa, fb, fe, fv, mlp1, mlp2):
    s = edge_index[0]
    r = edge_index[1]
    epad = _EP - _E
    sidx = jnp.concatenate([s, jnp.full((epad,), _N, jnp.int32)]).reshape(_R, _W)
    ridx = jnp.concatenate([r, jnp.full((epad,), _N, jnp.int32)]).reshape(_R, _W)
    x_p = jnp.pad(x, ((0, _NP - _N), (0, 0)))
    ea_p = jnp.pad(edge_attr, ((0, epad), (0, 0)))
    nt_p = jnp.pad(node_type, ((0, _NP - _N), (0, 3)))
    zeros_n = jnp.zeros((_NP, 64), _F32)
    zeros_n16 = jnp.zeros((_NP, 16), _F32)

    h0 = _embed_nodes(x_p, fa)
    e0 = _embed_edges(ea_p, fb)

    hs0, hr0 = _gather_two(h0, sidx, ridx)
    e1, upd = _edge_step1(hs0, hr0, e0, fe, fv)
    p0, p1 = _scatter_add64(upd.reshape(_R, _W, 64), ridx, zeros_n)
    h1 = _combine_h(h0, p0, p1)

    hs1, hr1 = _gather_two(h1, sidx, ridx)
    fij = _edge_step2_force(hs1, hr1, e1, fe, mlp1)

    ar0, ar1, as0, as1 = _scatter_add16_two(fij.reshape(_R, _W, 16), ridx, sidx,
                                            zeros_n16)
    ai_pad, gamma_pad = _final_node(ar0, ar1, as0, as1, nt_p, mlp2)
    return ai_pad[:_N, :3], gamma_pad[:_N, :1]


# speedup vs baseline: 1.5266x; 1.5266x over previous
import functools

import jax
import jax.numpy as jnp
from jax import lax
from jax.experimental import pallas as pl
from jax.experimental.pallas import tpu as pltpu
from jax.experimental.pallas import tpu_sc as plsc

_N = 10000
_NP = 10240
_E = 320000
_W = 128
_EP = 327680
_R = _EP // _W
_NW = 32
_RPW = _R // _NW
_CH = 8
_NCH = _RPW // _CH
_GSUB = 4
_SSUB = 2

_BLK_E = 2048
_GRID_E = _EP // _BLK_E
_BLK_N = 1024
_GRID_N = _NP // _BLK_N

_F32 = jnp.float32


def _sp(x):
    return jnp.maximum(x, 0.0) + jnp.log1p(jnp.exp(-jnp.abs(x)))


def _dot(a, w):
    return jax.lax.dot_general(a, w, (((1,), (0,)), ((), ())),
                               preferred_element_type=_F32)


def _wspec(shape):
    return pl.BlockSpec(shape, lambda i: (0, 0))


def _rowspec(blk, width):
    return pl.BlockSpec((blk, width), lambda i: (i, 0))


def _embed_nodes(x_p, fa):
    (w1, b1), (w2, b2) = fa
    w2p = jnp.pad(w2, ((0, 0), (0, 64)))
    b2p = jnp.pad(b2.reshape(1, 64), ((0, 0), (0, 64)))

    def body(x_ref, w1_ref, b1_ref, w2_ref, b2_ref, o_ref):
        t = _sp(_dot(x_ref[...], w1_ref[...]) + b1_ref[...])
        o_ref[...] = _dot(t, w2_ref[...]) + b2_ref[...]

    return pl.pallas_call(
        body,
        grid=(_GRID_N,),
        in_specs=[_rowspec(_BLK_N, 128), _wspec((128, 64)), _wspec((1, 64)),
                  _wspec((64, 128)), _wspec((1, 128))],
        out_specs=_rowspec(_BLK_N, 128),
        out_shape=jax.ShapeDtypeStruct((_NP, 128), _F32),
    )(x_p, w1, b1.reshape(1, 64), w2p, b2p)


def _embed_edges(ea_p, fb):
    (w1, b1), (w2, b2) = fb

    def body(a_ref, w1_ref, b1_ref, w2_ref, b2_ref, o_ref):
        t = _sp(_dot(a_ref[...], w1_ref[...]) + b1_ref[...])
        o_ref[...] = _dot(t, w2_ref[...]) + b2_ref[...]

    return pl.pallas_call(
        body,
        grid=(_GRID_E,),
        in_specs=[_rowspec(_BLK_E, 16), _wspec((16, 64)), _wspec((1, 64)),
                  _wspec((64, 64)), _wspec((1, 64))],
        out_specs=_rowspec(_BLK_E, 64),
        out_shape=jax.ShapeDtypeStruct((_EP, 64), _F32),
    )(ea_p, w1, b1.reshape(1, 64), w2, b2.reshape(1, 64))


def _edge_step1(hs, hr, e0, fe, fv):
    (we1, be1), (we2, be2), (we3, be3) = fe
    (wv1, bv1), (wv2, bv2), (wv3, bv3) = fv
    we1p = jnp.pad(we1, ((0, 64), (0, 0)))
    wva = jnp.pad(wv1[:64], ((0, 64), (0, 0)))
    wvb = wv1[64:]
    wv3p = jnp.pad(wv3, ((0, 0), (0, 64)))
    bv3p = jnp.pad(bv3.reshape(1, 64), ((0, 0), (0, 64)))

    def body(hs_ref, hr_ref, e0_ref,
             we1_r, be1_r, we2_r, be2_r, we3_r, be3_r,
             wva_r, wvb_r, bv1_r, wv2_r, bv2_r, wv3_r, bv3_r,
             e1_o, upd_o):
        prod = hs_ref[...] * hr_ref[...]
        t = _sp(_dot(prod, we1_r[...]) + be1_r[...])
        t = _sp(_dot(t, we2_r[...]) + be2_r[...])
        e1 = _dot(t, we3_r[...]) + be3_r[...] + e0_ref[...]
        e1_o[...] = e1
        u = _sp(_dot(hr_ref[...], wva_r[...]) + _dot(e1, wvb_r[...]) + bv1_r[...])
        u = _sp(_dot(u, wv2_r[...]) + bv2_r[...])
        upd_o[...] = _dot(u, wv3_r[...]) + bv3_r[...]

    return pl.pallas_call(
        body,
        grid=(_GRID_E,),
        in_specs=[_rowspec(_BLK_E, 128), _rowspec(_BLK_E, 128),
                  _rowspec(_BLK_E, 64)] +
                 [_wspec((128, 64)), _wspec((1, 64)),
                  _wspec((64, 64)), _wspec((1, 64)),
                  _wspec((64, 64)), _wspec((1, 64))] +
                 [_wspec((128, 64)), _wspec((64, 64)), _wspec((1, 64)),
                  _wspec((64, 64)), _wspec((1, 64)),
                  _wspec((64, 128)), _wspec((1, 128))],
        out_specs=[_rowspec(_BLK_E, 64), _rowspec(_BLK_E, 128)],
        out_shape=[jax.ShapeDtypeStruct((_EP, 64), _F32),
                   jax.ShapeDtypeStruct((_EP, 128), _F32)],
    )(hs, hr, e0,
      we1p, be1.reshape(1, 64), we2, be2.reshape(1, 64), we3, be3.reshape(1, 64),
      wva, wvb, bv1.reshape(1, 64), wv2, bv2.reshape(1, 64), wv3p, bv3p)


def _edge_step2_force(hs, hr, e1, fe, mlp1):
    (we1, be1), (we2, be2), (we3, be3) = fe
    (wm1, bm1), (wm2, bm2), (wm3, bm3) = mlp1
    we1p = jnp.pad(we1, ((0, 64), (0, 0)))
    wm3p = jnp.pad(wm3, ((0, 0), (0, 125)))
    bm3p = jnp.pad(bm3.reshape(1, 3), ((0, 0), (0, 125)))

    def body(hs_ref, hr_ref, e1_ref,
             we1_r, be1_r, we2_r, be2_r, we3_r, be3_r,
             wm1_r, bm1_r, wm2_r, bm2_r, wm3_r, bm3_r,
             fij_o):
        prod = hs_ref[...] * hr_ref[...]
        t = _sp(_dot(prod, we1_r[...]) + be1_r[...])
        t = _sp(_dot(t, we2_r[...]) + be2_r[...])
        e2 = _dot(t, we3_r[...]) + be3_r[...] + e1_ref[...]
        f = _sp(_dot(e2, wm1_r[...]) + bm1_r[...])
        f = _sp(_dot(f, wm2_r[...]) + bm2_r[...])
        fij_o[...] = _dot(f, wm3_r[...]) + bm3_r[...]

    return pl.pallas_call(
        body,
        grid=(_GRID_E,),
        in_specs=[_rowspec(_BLK_E, 128), _rowspec(_BLK_E, 128),
                  _rowspec(_BLK_E, 64)] +
                 [_wspec((128, 64)), _wspec((1, 64))] +
                 [_wspec((64, 64)), _wspec((1, 64))] * 4 +
                 [_wspec((64, 128)), _wspec((1, 128))],
        out_specs=_rowspec(_BLK_E, 128),
        out_shape=jax.ShapeDtypeStruct((_EP, 128), _F32),
    )(hs, hr, e1,
      we1p, be1.reshape(1, 64), we2, be2.reshape(1, 64), we3, be3.reshape(1, 64),
      wm1, bm1.reshape(1, 64), wm2, bm2.reshape(1, 64), wm3p, bm3p)


def _combine_h(h0, p0, p1):
    def body(a_ref, b_ref, c_ref, o_ref):
        o_ref[...] = a_ref[...] + b_ref[...] + c_ref[...]

    return pl.pallas_call(
        body,
        grid=(_GRID_N,),
        in_specs=[_rowspec(_BLK_N, 128)] * 3,
        out_specs=_rowspec(_BLK_N, 128),
        out_shape=jax.ShapeDtypeStruct((_NP, 128), _F32),
    )(h0, p0, p1)


def _final_node(pr, ps, nt_p, mlp2):
    (w1, b1), (w2, b2), (w3, b3) = mlp2
    w1p = jnp.pad(w1, ((0, 3), (0, 3)))
    b1p = jnp.pad(b1.reshape(1, 5), ((0, 0), (0, 3)))
    w2p = jnp.pad(w2, ((0, 3), (0, 3)))
    b2p = jnp.pad(b2.reshape(1, 5), ((0, 0), (0, 3)))
    w3p = jnp.pad(w3, ((0, 3), (0, 7)))
    b3p = jnp.pad(b3.reshape(1, 1), ((0, 0), (0, 7)))

    def body(pr_r, ps_r, nt_r,
             w1_r, b1_r, w2_r, b2_r, w3_r, b3_r,
             ai_o, g_o):
        ai_o[...] = pr_r[...] - ps_r[...]
        g = _sp(_dot(nt_r[...], w1_r[...]) + b1_r[...])
        g = _sp(_dot(g, w2_r[...]) + b2_r[...])
        g_o[...] = _sp(_dot(g, w3_r[...]) + b3_r[...])

    return pl.pallas_call(
        body,
        grid=(_GRID_N,),
        in_specs=[_rowspec(_BLK_N, 128)] * 2 + [_rowspec(_BLK_N, 8)] +
                 [_wspec((8, 8)), _wspec((1, 8))] * 3,
        out_specs=[_rowspec(_BLK_N, 128), _rowspec(_BLK_N, 8)],
        out_shape=[jax.ShapeDtypeStruct((_NP, 128), _F32),
                   jax.ShapeDtypeStruct((_NP, 8), _F32)],
    )(pr, ps, nt_p, w1p, b1p, w2p, b2p, w3p, b3p)


_MESH = plsc.VectorSubcoreMesh(core_axis_name="c", subcore_axis_name="s")


def _gather_two(h_pad, sidx, ridx):
    @functools.partial(
        pl.kernel,
        out_type=[jax.ShapeDtypeStruct((_R, _W, 128), _F32)] * 2,
        mesh=_MESH,
        scratch_types=[pltpu.VMEM((_CH, _W), jnp.int32),
                       pltpu.VMEM((_GSUB, _W, 128), _F32),
                       pltpu.SemaphoreType.DMA],
    )
    def k(h_hbm, s_hbm, r_hbm, hs_out, hr_out, idx_v, rows_v, sem):
        wid = lax.axis_index("s") * 2 + lax.axis_index("c")
        base = wid * _RPW

        def run(idx_hbm, out_hbm):
            def chunk(ci, carry):
                row0 = base + ci * _CH
                pltpu.sync_copy(idx_hbm.at[pl.ds(row0, _CH)], idx_v)
                for g in range(_CH // _GSUB):
                    cps = [pltpu.async_copy(h_hbm.at[idx_v.at[g * _GSUB + j]],
                                            rows_v.at[j], sem)
                           for j in range(_GSUB)]
                    for cp in cps:
                        cp.wait()
                    pltpu.sync_copy(rows_v,
                                    out_hbm.at[pl.ds(row0 + g * _GSUB, _GSUB)])
                return carry
            lax.fori_loop(0, _NCH, chunk, 0)

        run(s_hbm, hs_out)
        run(r_hbm, hr_out)

    hs, hr = k(h_pad, sidx, ridx)
    return hs.reshape(_EP, 128), hr.reshape(_EP, 128)


def _scatter_add64(vals, ridx, zeros_n):
    @functools.partial(
        pl.kernel,
        out_type=[jax.ShapeDtypeStruct((_NP, 128), _F32)] * 2,
        mesh=_MESH,
        scratch_types=[pltpu.VMEM((_CH, _W), jnp.int32),
                       pltpu.VMEM((_SSUB, _W, 128), _F32),
                       pltpu.VMEM_SHARED((_NP, 128), _F32),
                       pltpu.SemaphoreType.DMA],
    )
    def k(v_hbm, i_hbm, z_hbm, p0_out, p1_out, idx_v, rows_v, acc, sem):
        cid = lax.axis_index("c")
        sid = lax.axis_index("s")
        nsl = _NP // 16
        pltpu.sync_copy(z_hbm.at[pl.ds(sid * nsl, nsl)],
                        acc.at[pl.ds(sid * nsl, nsl)])
        plsc.subcore_barrier()
        base = cid * (_R // 2) + sid * _RPW

        def chunk(ci, carry):
            row0 = base + ci * _CH
            pltpu.sync_copy(i_hbm.at[pl.ds(row0, _CH)], idx_v)
            for g in range(_CH // _SSUB):
                pltpu.sync_copy(v_hbm.at[pl.ds(row0 + g * _SSUB, _SSUB)],
                                rows_v)
                for j in range(_SSUB):
                    pltpu.sync_copy(rows_v.at[j],
                                    acc.at[idx_v.at[g * _SSUB + j]], add=True)
            return carry

        lax.fori_loop(0, _NCH, chunk, 0)
        plsc.subcore_barrier()

        @pl.when(cid == 0)
        def _():
            pltpu.sync_copy(acc.at[pl.ds(sid * nsl, nsl)],
                            p0_out.at[pl.ds(sid * nsl, nsl)])

        @pl.when(cid == 1)
        def _():
            pltpu.sync_copy(acc.at[pl.ds(sid * nsl, nsl)],
                            p1_out.at[pl.ds(sid * nsl, nsl)])

    return k(vals, ridx, zeros_n)


def _scatter_fij(vals, ridx, sidx, zeros_n):
    @functools.partial(
        pl.kernel,
        out_type=[jax.ShapeDtypeStruct((_NP, 128), _F32)] * 2,
        mesh=_MESH,
        scratch_types=[pltpu.VMEM((_CH, _W), jnp.int32),
                       pltpu.VMEM((_SSUB, _W, 128), _F32),
                       pltpu.VMEM_SHARED((_NP, 128), _F32),
                       pltpu.SemaphoreType.DMA],
    )
    def k(v_hbm, ri_hbm, si_hbm, z_hbm, pr_out, ps_out,
          idx_v, rows_v, acc, sem):
        cid = lax.axis_index("c")
        sid = lax.axis_index("s")
        nsl = _NP // 16
        pltpu.sync_copy(z_hbm.at[pl.ds(sid * nsl, nsl)],
                        acc.at[pl.ds(sid * nsl, nsl)])
        plsc.subcore_barrier()
        rpt = _R // 16
        base = sid * rpt

        def make_chunk(idx_hbm):
            def chunk(ci, carry):
                row0 = base + ci * _CH
                pltpu.sync_copy(idx_hbm.at[pl.ds(row0, _CH)], idx_v)
                for g in range(_CH // _SSUB):
                    pltpu.sync_copy(v_hbm.at[pl.ds(row0 + g * _SSUB, _SSUB)],
                                    rows_v)
                    for j in range(_SSUB):
                        pltpu.sync_copy(rows_v.at[j],
                                        acc.at[idx_v.at[g * _SSUB + j]],
                                        add=True)
                return carry
            return chunk

        @pl.when(cid == 0)
        def _():
            lax.fori_loop(0, rpt // _CH, make_chunk(ri_hbm), 0)

        @pl.when(cid == 1)
        def _():
            lax.fori_loop(0, rpt // _CH, make_chunk(si_hbm), 0)

        plsc.subcore_barrier()

        @pl.when(cid == 0)
        def _():
            pltpu.sync_copy(acc.at[pl.ds(sid * nsl, nsl)],
                            pr_out.at[pl.ds(sid * nsl, nsl)])

        @pl.when(cid == 1)
        def _():
            pltpu.sync_copy(acc.at[pl.ds(sid * nsl, nsl)],
                            ps_out.at[pl.ds(sid * nsl, nsl)])

    return k(vals, ridx, sidx, zeros_n)


def kernel(x, edge_attr, node_type, edge_index, fa, fb, fe, fv, mlp1, mlp2):
    s = edge_index[0]
    r = edge_index[1]
    epad = _EP - _E
    sidx = jnp.concatenate([s, jnp.full((epad,), _N, jnp.int32)]).reshape(_R, _W)
    ridx = jnp.concatenate([r, jnp.full((epad,), _N, jnp.int32)]).reshape(_R, _W)
    x_p = jnp.pad(x, ((0, _NP - _N), (0, 0)))
    ea_p = jnp.pad(edge_attr, ((0, epad), (0, 0)))
    nt_p = jnp.pad(node_type, ((0, _NP - _N), (0, 3)))
    zeros_n = jnp.zeros((_NP, 128), _F32)

    h0 = _embed_nodes(x_p, fa)
    e0 = _embed_edges(ea_p, fb)

    hs0, hr0 = _gather_two(h0, sidx, ridx)
    e1, upd = _edge_step1(hs0, hr0, e0, fe, fv)
    p0, p1 = _scatter_add64(upd.reshape(_R, _W, 128), ridx, zeros_n)
    h1 = _combine_h(h0, p0, p1)

    hs1, hr1 = _gather_two(h1, sidx, ridx)
    fij = _edge_step2_force(hs1, hr1, e1, fe, mlp1)

    pr, ps = _scatter_fij(fij.reshape(_R, _W, 128), ridx, sidx, zeros_n)
    ai_pad, gamma_pad = _final_node(pr, ps, nt_p, mlp2)
    return ai_pad[:_N, :3], gamma_pad[:_N, :1]


# speedup vs baseline: 1.5961x; 1.0455x over previous
import functools

import jax
import jax.numpy as jnp
from jax import lax
from jax.experimental import pallas as pl
from jax.experimental.pallas import tpu as pltpu
from jax.experimental.pallas import tpu_sc as plsc

_N = 10000
_NP = 10240
_E = 320000
_W = 128
_EP = 327680
_R = _EP // _W
_NW = 32
_RPW = _R // _NW
_CH = 8
_NCH = _RPW // _CH
_GSUB = 4
_SSUB = 2

_BLK_E = 2048
_GRID_E = _EP // _BLK_E
_BLK_N = 1024
_GRID_N = _NP // _BLK_N

_F32 = jnp.float32


def _sp(x):
    return jnp.maximum(x, 0.0) + jnp.log1p(jnp.exp(-jnp.abs(x)))


def _dot(a, w):
    return jax.lax.dot_general(a, w, (((1,), (0,)), ((), ())),
                               preferred_element_type=_F32)


def _wspec(shape):
    return pl.BlockSpec(shape, lambda i: (0, 0))


def _rowspec(blk, width):
    return pl.BlockSpec((blk, width), lambda i: (i, 0))


def _embed_nodes(x_p, fa):
    (w1, b1), (w2, b2) = fa
    w2p = jnp.pad(w2, ((0, 0), (0, 64)))
    b2p = jnp.pad(b2.reshape(1, 64), ((0, 0), (0, 64)))

    def body(x_ref, w1_ref, b1_ref, w2_ref, b2_ref, o_ref):
        t = _sp(_dot(x_ref[...], w1_ref[...]) + b1_ref[...])
        o_ref[...] = _dot(t, w2_ref[...]) + b2_ref[...]

    return pl.pallas_call(
        body,
        grid=(_GRID_N,),
        in_specs=[_rowspec(_BLK_N, 128), _wspec((128, 64)), _wspec((1, 64)),
                  _wspec((64, 128)), _wspec((1, 128))],
        out_specs=_rowspec(_BLK_N, 128),
        out_shape=jax.ShapeDtypeStruct((_NP, 128), _F32),
    )(x_p, w1, b1.reshape(1, 64), w2p, b2p)


def _embed_edges(ea_p, fb):
    (w1, b1), (w2, b2) = fb

    def body(a_ref, w1_ref, b1_ref, w2_ref, b2_ref, o_ref):
        t = _sp(_dot(a_ref[...], w1_ref[...]) + b1_ref[...])
        o_ref[...] = _dot(t, w2_ref[...]) + b2_ref[...]

    return pl.pallas_call(
        body,
        grid=(_GRID_E,),
        in_specs=[_rowspec(_BLK_E, 16), _wspec((16, 64)), _wspec((1, 64)),
                  _wspec((64, 64)), _wspec((1, 64))],
        out_specs=_rowspec(_BLK_E, 64),
        out_shape=jax.ShapeDtypeStruct((_EP, 64), _F32),
    )(ea_p, w1, b1.reshape(1, 64), w2, b2.reshape(1, 64))


def _edge_step1(hs, hr, e0, fe, fv):
    (we1, be1), (we2, be2), (we3, be3) = fe
    (wv1, bv1), (wv2, bv2), (wv3, bv3) = fv
    we1p = jnp.pad(we1, ((0, 64), (0, 0)))
    wva = jnp.pad(wv1[:64], ((0, 64), (0, 0)))
    wvb = wv1[64:]
    wv3p = jnp.pad(wv3, ((0, 0), (0, 64)))
    bv3p = jnp.pad(bv3.reshape(1, 64), ((0, 0), (0, 64)))

    def body(hs_ref, hr_ref, e0_ref,
             we1_r, be1_r, we2_r, be2_r, we3_r, be3_r,
             wva_r, wvb_r, bv1_r, wv2_r, bv2_r, wv3_r, bv3_r,
             e1_o, upd_o):
        prod = hs_ref[...] * hr_ref[...]
        t = _sp(_dot(prod, we1_r[...]) + be1_r[...])
        t = _sp(_dot(t, we2_r[...]) + be2_r[...])
        e1 = _dot(t, we3_r[...]) + be3_r[...] + e0_ref[...]
        e1_o[...] = e1
        u = _sp(_dot(hr_ref[...], wva_r[...]) + _dot(e1, wvb_r[...]) + bv1_r[...])
        u = _sp(_dot(u, wv2_r[...]) + bv2_r[...])
        upd_o[...] = _dot(u, wv3_r[...]) + bv3_r[...]

    return pl.pallas_call(
        body,
        grid=(_GRID_E,),
        in_specs=[_rowspec(_BLK_E, 128), _rowspec(_BLK_E, 128),
                  _rowspec(_BLK_E, 64)] +
                 [_wspec((128, 64)), _wspec((1, 64)),
                  _wspec((64, 64)), _wspec((1, 64)),
                  _wspec((64, 64)), _wspec((1, 64))] +
                 [_wspec((128, 64)), _wspec((64, 64)), _wspec((1, 64)),
                  _wspec((64, 64)), _wspec((1, 64)),
                  _wspec((64, 128)), _wspec((1, 128))],
        out_specs=[_rowspec(_BLK_E, 64), _rowspec(_BLK_E, 128)],
        out_shape=[jax.ShapeDtypeStruct((_EP, 64), _F32),
                   jax.ShapeDtypeStruct((_EP, 128), _F32)],
    )(hs, hr, e0,
      we1p, be1.reshape(1, 64), we2, be2.reshape(1, 64), we3, be3.reshape(1, 64),
      wva, wvb, bv1.reshape(1, 64), wv2, bv2.reshape(1, 64), wv3p, bv3p)


def _edge_step2_force(hs, hr, e1, fe, mlp1):
    (we1, be1), (we2, be2), (we3, be3) = fe
    (wm1, bm1), (wm2, bm2), (wm3, bm3) = mlp1
    we1p = jnp.pad(we1, ((0, 64), (0, 0)))
    wm3p = jnp.pad(wm3, ((0, 0), (0, 125)))
    bm3p = jnp.pad(bm3.reshape(1, 3), ((0, 0), (0, 125)))

    def body(hs_ref, hr_ref, e1_ref,
             we1_r, be1_r, we2_r, be2_r, we3_r, be3_r,
             wm1_r, bm1_r, wm2_r, bm2_r, wm3_r, bm3_r,
             fij_o):
        prod = hs_ref[...] * hr_ref[...]
        t = _sp(_dot(prod, we1_r[...]) + be1_r[...])
        t = _sp(_dot(t, we2_r[...]) + be2_r[...])
        e2 = _dot(t, we3_r[...]) + be3_r[...] + e1_ref[...]
        f = _sp(_dot(e2, wm1_r[...]) + bm1_r[...])
        f = _sp(_dot(f, wm2_r[...]) + bm2_r[...])
        fij_o[...] = _dot(f, wm3_r[...]) + bm3_r[...]

    return pl.pallas_call(
        body,
        grid=(_GRID_E,),
        in_specs=[_rowspec(_BLK_E, 128), _rowspec(_BLK_E, 128),
                  _rowspec(_BLK_E, 64)] +
                 [_wspec((128, 64)), _wspec((1, 64))] +
                 [_wspec((64, 64)), _wspec((1, 64))] * 4 +
                 [_wspec((64, 128)), _wspec((1, 128))],
        out_specs=_rowspec(_BLK_E, 128),
        out_shape=jax.ShapeDtypeStruct((_EP, 128), _F32),
    )(hs, hr, e1,
      we1p, be1.reshape(1, 64), we2, be2.reshape(1, 64), we3, be3.reshape(1, 64),
      wm1, bm1.reshape(1, 64), wm2, bm2.reshape(1, 64), wm3p, bm3p)


def _combine_h(h0, p0, p1):
    def body(a_ref, b_ref, c_ref, o_ref):
        o_ref[...] = a_ref[...] + b_ref[...] + c_ref[...]

    return pl.pallas_call(
        body,
        grid=(_GRID_N,),
        in_specs=[_rowspec(_BLK_N, 128)] * 3,
        out_specs=_rowspec(_BLK_N, 128),
        out_shape=jax.ShapeDtypeStruct((_NP, 128), _F32),
    )(h0, p0, p1)


def _final_node(pr, ps, nt_p, mlp2):
    (w1, b1), (w2, b2), (w3, b3) = mlp2
    w1p = jnp.pad(w1, ((0, 3), (0, 3)))
    b1p = jnp.pad(b1.reshape(1, 5), ((0, 0), (0, 3)))
    w2p = jnp.pad(w2, ((0, 3), (0, 3)))
    b2p = jnp.pad(b2.reshape(1, 5), ((0, 0), (0, 3)))
    w3p = jnp.pad(w3, ((0, 3), (0, 7)))
    b3p = jnp.pad(b3.reshape(1, 1), ((0, 0), (0, 7)))

    def body(pr_r, ps_r, nt_r,
             w1_r, b1_r, w2_r, b2_r, w3_r, b3_r,
             ai_o, g_o):
        ai_o[...] = pr_r[...] - ps_r[...]
        g = _sp(_dot(nt_r[...], w1_r[...]) + b1_r[...])
        g = _sp(_dot(g, w2_r[...]) + b2_r[...])
        g_o[...] = _sp(_dot(g, w3_r[...]) + b3_r[...])

    return pl.pallas_call(
        body,
        grid=(_GRID_N,),
        in_specs=[_rowspec(_BLK_N, 128)] * 2 + [_rowspec(_BLK_N, 8)] +
                 [_wspec((8, 8)), _wspec((1, 8))] * 3,
        out_specs=[_rowspec(_BLK_N, 128), _rowspec(_BLK_N, 8)],
        out_shape=[jax.ShapeDtypeStruct((_NP, 128), _F32),
                   jax.ShapeDtypeStruct((_NP, 8), _F32)],
    )(pr, ps, nt_p, w1p, b1p, w2p, b2p, w3p, b3p)


_MESH = plsc.VectorSubcoreMesh(core_axis_name="c", subcore_axis_name="s")


def _gather_two(h_pad, sidx, ridx):
    nbuf = 4

    @functools.partial(
        pl.kernel,
        out_type=[jax.ShapeDtypeStruct((_R, _W, 128), _F32)] * 2,
        mesh=_MESH,
        scratch_types=[pltpu.VMEM((_CH, _W), jnp.int32)] +
                      [pltpu.VMEM((_W, 128), _F32)] * nbuf +
                      [pltpu.SemaphoreType.DMA] * (2 * nbuf),
    )
    def k(h_hbm, s_hbm, r_hbm, hs_out, hr_out, idx_v, *scr):
        bufs = scr[:nbuf]
        sem_g = scr[nbuf:2 * nbuf]
        sem_w = scr[2 * nbuf:]
        wid = lax.axis_index("s") * 2 + lax.axis_index("c")
        base = wid * _RPW

        def run(idx_hbm, out_hbm):
            def wait_w(b):
                pltpu.make_async_copy(bufs[b], out_hbm.at[0], sem_w[b]).wait()

            def chunk(ci, carry):
                row0 = base + ci * _CH
                pltpu.sync_copy(idx_hbm.at[pl.ds(row0, _CH)], idx_v)
                for g in range(_CH):
                    b = g % nbuf
                    if g < nbuf:
                        @pl.when(ci > 0)
                        def _():
                            wait_w(b)
                    else:
                        wait_w(b)
                    pltpu.async_copy(h_hbm.at[idx_v.at[g]], bufs[b], sem_g[b])
                    if g >= 2:
                        pb = (g - 2) % nbuf
                        pltpu.make_async_copy(out_hbm.at[0], bufs[pb],
                                              sem_g[pb]).wait()
                        pltpu.async_copy(bufs[pb], out_hbm.at[row0 + g - 2],
                                         sem_w[pb])
                for g in (_CH - 2, _CH - 1):
                    pb = g % nbuf
                    pltpu.make_async_copy(out_hbm.at[0], bufs[pb],
                                          sem_g[pb]).wait()
                    pltpu.async_copy(bufs[pb], out_hbm.at[row0 + g],
                                     sem_w[pb])
                return carry

            lax.fori_loop(0, _NCH, chunk, 0)
            for b in range(nbuf):
                wait_w(b)

        run(s_hbm, hs_out)
        run(r_hbm, hr_out)

    hs, hr = k(h_pad, sidx, ridx)
    return hs.reshape(_EP, 128), hr.reshape(_EP, 128)


def _scatter_add64(vals, ridx, zeros_n):
    @functools.partial(
        pl.kernel,
        out_type=[jax.ShapeDtypeStruct((_NP, 128), _F32)] * 2,
        mesh=_MESH,
        scratch_types=[pltpu.VMEM((_CH, _W), jnp.int32),
                       pltpu.VMEM((_SSUB, _W, 128), _F32),
                       pltpu.VMEM_SHARED((_NP, 128), _F32),
                       pltpu.SemaphoreType.DMA],
    )
    def k(v_hbm, i_hbm, z_hbm, p0_out, p1_out, idx_v, rows_v, acc, sem):
        cid = lax.axis_index("c")
        sid = lax.axis_index("s")
        nsl = _NP // 16
        pltpu.sync_copy(z_hbm.at[pl.ds(sid * nsl, nsl)],
                        acc.at[pl.ds(sid * nsl, nsl)])
        plsc.subcore_barrier()
        base = cid * (_R // 2) + sid * _RPW

        def chunk(ci, carry):
            row0 = base + ci * _CH
            pltpu.sync_copy(i_hbm.at[pl.ds(row0, _CH)], idx_v)
            for g in range(_CH // _SSUB):
                pltpu.sync_copy(v_hbm.at[pl.ds(row0 + g * _SSUB, _SSUB)],
                                rows_v)
                for j in range(_SSUB):
                    pltpu.sync_copy(rows_v.at[j],
                                    acc.at[idx_v.at[g * _SSUB + j]], add=True)
            return carry

        lax.fori_loop(0, _NCH, chunk, 0)
        plsc.subcore_barrier()

        @pl.when(cid == 0)
        def _():
            pltpu.sync_copy(acc.at[pl.ds(sid * nsl, nsl)],
                            p0_out.at[pl.ds(sid * nsl, nsl)])

        @pl.when(cid == 1)
        def _():
            pltpu.sync_copy(acc.at[pl.ds(sid * nsl, nsl)],
                            p1_out.at[pl.ds(sid * nsl, nsl)])

    return k(vals, ridx, zeros_n)


def _scatter_fij(vals, ridx, sidx, zeros_n):
    @functools.partial(
        pl.kernel,
        out_type=[jax.ShapeDtypeStruct((_NP, 128), _F32)] * 2,
        mesh=_MESH,
        scratch_types=[pltpu.VMEM((_CH, _W), jnp.int32),
                       pltpu.VMEM((_SSUB, _W, 128), _F32),
                       pltpu.VMEM_SHARED((_NP, 128), _F32),
                       pltpu.SemaphoreType.DMA],
    )
    def k(v_hbm, ri_hbm, si_hbm, z_hbm, pr_out, ps_out,
          idx_v, rows_v, acc, sem):
        cid = lax.axis_index("c")
        sid = lax.axis_index("s")
        nsl = _NP // 16
        pltpu.sync_copy(z_hbm.at[pl.ds(sid * nsl, nsl)],
                        acc.at[pl.ds(sid * nsl, nsl)])
        plsc.subcore_barrier()
        rpt = _R // 16
        base = sid * rpt

        def make_chunk(idx_hbm):
            def chunk(ci, carry):
                row0 = base + ci * _CH
                pltpu.sync_copy(idx_hbm.at[pl.ds(row0, _CH)], idx_v)
                for g in range(_CH // _SSUB):
                    pltpu.sync_copy(v_hbm.at[pl.ds(row0 + g * _SSUB, _SSUB)],
                                    rows_v)
                    for j in range(_SSUB):
                        pltpu.sync_copy(rows_v.at[j],
                                        acc.at[idx_v.at[g * _SSUB + j]],
                                        add=True)
                return carry
            return chunk

        @pl.when(cid == 0)
        def _():
            lax.fori_loop(0, rpt // _CH, make_chunk(ri_hbm), 0)

        @pl.when(cid == 1)
        def _():
            lax.fori_loop(0, rpt // _CH, make_chunk(si_hbm), 0)

        plsc.subcore_barrier()

        @pl.when(cid == 0)
        def _():
            pltpu.sync_copy(acc.at[pl.ds(sid * nsl, nsl)],
                            pr_out.at[pl.ds(sid * nsl, nsl)])

        @pl.when(cid == 1)
        def _():
            pltpu.sync_copy(acc.at[pl.ds(sid * nsl, nsl)],
                            ps_out.at[pl.ds(sid * nsl, nsl)])

    return k(vals, ridx, sidx, zeros_n)


def kernel(x, edge_attr, node_type, edge_index, fa, fb, fe, fv, mlp1, mlp2):
    s = edge_index[0]
    r = edge_index[1]
    epad = _EP - _E
    sidx = jnp.concatenate([s, jnp.full((epad,), _N, jnp.int32)]).reshape(_R, _W)
    ridx = jnp.concatenate([r, jnp.full((epad,), _N, jnp.int32)]).reshape(_R, _W)
    x_p = jnp.pad(x, ((0, _NP - _N), (0, 0)))
    ea_p = jnp.pad(edge_attr, ((0, epad), (0, 0)))
    nt_p = jnp.pad(node_type, ((0, _NP - _N), (0, 3)))
    zeros_n = jnp.zeros((_NP, 128), _F32)

    h0 = _embed_nodes(x_p, fa)
    e0 = _embed_edges(ea_p, fb)

    hs0, hr0 = _gather_two(h0, sidx, ridx)
    e1, upd = _edge_step1(hs0, hr0, e0, fe, fv)
    p0, p1 = _scatter_add64(upd.reshape(_R, _W, 128), ridx, zeros_n)
    h1 = _combine_h(h0, p0, p1)

    hs1, hr1 = _gather_two(h1, sidx, ridx)
    fij = _edge_step2_force(hs1, hr1, e1, fe, mlp1)

    pr, ps = _scatter_fij(fij.reshape(_R, _W, 128), ridx, sidx, zeros_n)
    ai_pad, gamma_pad = _final_node(pr, ps, nt_p, mlp2)
    return ai_pad[:_N, :3], gamma_pad[:_N, :1]


# speedup vs baseline: 2.7620x; 1.7305x over previous
import functools

import jax
import jax.numpy as jnp
from jax import lax
from jax.experimental import pallas as pl
from jax.experimental.pallas import tpu as pltpu
from jax.experimental.pallas import tpu_sc as plsc

_N = 10000
_NP = 10240
_E = 320000
_W = 128
_EP = 327680
_R = _EP // _W
_NW = 32
_RPW = _R // _NW
_CH = 8
_NCH = _RPW // _CH
_GSUB = 4
_SSUB = 2

_BLK_E = 2048
_GRID_E = _EP // _BLK_E
_BLK_N = 1024
_GRID_N = _NP // _BLK_N

_F32 = jnp.float32


def _sp(x):
    return jnp.maximum(x, 0.0) + jnp.log1p(jnp.exp(-jnp.abs(x)))


def _dot(a, w):
    return jax.lax.dot_general(a, w, (((1,), (0,)), ((), ())),
                               preferred_element_type=_F32)


def _wspec(shape):
    return pl.BlockSpec(shape, lambda i: (0, 0))


def _rowspec(blk, width):
    return pl.BlockSpec((blk, width), lambda i: (i, 0))


def _embed_nodes(x_p, fa):
    (w1, b1), (w2, b2) = fa
    w2p = jnp.pad(w2, ((0, 0), (0, 64)))
    b2p = jnp.pad(b2.reshape(1, 64), ((0, 0), (0, 64)))

    def body(x_ref, w1_ref, b1_ref, w2_ref, b2_ref, o_ref):
        t = _sp(_dot(x_ref[...], w1_ref[...]) + b1_ref[...])
        o_ref[...] = _dot(t, w2_ref[...]) + b2_ref[...]

    return pl.pallas_call(
        body,
        grid=(_GRID_N,),
        in_specs=[_rowspec(_BLK_N, 128), _wspec((128, 64)), _wspec((1, 64)),
                  _wspec((64, 128)), _wspec((1, 128))],
        out_specs=_rowspec(_BLK_N, 128),
        out_shape=jax.ShapeDtypeStruct((_NP, 128), _F32),
    )(x_p, w1, b1.reshape(1, 64), w2p, b2p)


def _embed_edges(ea_p, fb):
    (w1, b1), (w2, b2) = fb

    def body(a_ref, w1_ref, b1_ref, w2_ref, b2_ref, o_ref):
        t = _sp(_dot(a_ref[...], w1_ref[...]) + b1_ref[...])
        o_ref[...] = _dot(t, w2_ref[...]) + b2_ref[...]

    return pl.pallas_call(
        body,
        grid=(_GRID_E,),
        in_specs=[_rowspec(_BLK_E, 16), _wspec((16, 64)), _wspec((1, 64)),
                  _wspec((64, 64)), _wspec((1, 64))],
        out_specs=_rowspec(_BLK_E, 64),
        out_shape=jax.ShapeDtypeStruct((_EP, 64), _F32),
    )(ea_p, w1, b1.reshape(1, 64), w2, b2.reshape(1, 64))


def _edge_step1(hs, hr, e0, fe, fv):
    (we1, be1), (we2, be2), (we3, be3) = fe
    (wv1, bv1), (wv2, bv2), (wv3, bv3) = fv
    we1p = jnp.pad(we1, ((0, 64), (0, 0)))
    wva = jnp.pad(wv1[:64], ((0, 64), (0, 0)))
    wvb = wv1[64:]
    wv3p = jnp.pad(wv3, ((0, 0), (0, 64)))
    bv3p = jnp.pad(bv3.reshape(1, 64), ((0, 0), (0, 64)))

    def body(hs_ref, hr_ref, e0_ref,
             we1_r, be1_r, we2_r, be2_r, we3_r, be3_r,
             wva_r, wvb_r, bv1_r, wv2_r, bv2_r, wv3_r, bv3_r,
             e1_o, upd_o):
        prod = hs_ref[...] * hr_ref[...]
        t = _sp(_dot(prod, we1_r[...]) + be1_r[...])
        t = _sp(_dot(t, we2_r[...]) + be2_r[...])
        e1 = _dot(t, we3_r[...]) + be3_r[...] + e0_ref[...]
        e1_o[...] = e1
        u = _sp(_dot(hr_ref[...], wva_r[...]) + _dot(e1, wvb_r[...]) + bv1_r[...])
        u = _sp(_dot(u, wv2_r[...]) + bv2_r[...])
        upd_o[...] = _dot(u, wv3_r[...]) + bv3_r[...]

    return pl.pallas_call(
        body,
        grid=(_GRID_E,),
        in_specs=[_rowspec(_BLK_E, 128), _rowspec(_BLK_E, 128),
                  _rowspec(_BLK_E, 64)] +
                 [_wspec((128, 64)), _wspec((1, 64)),
                  _wspec((64, 64)), _wspec((1, 64)),
                  _wspec((64, 64)), _wspec((1, 64))] +
                 [_wspec((128, 64)), _wspec((64, 64)), _wspec((1, 64)),
                  _wspec((64, 64)), _wspec((1, 64)),
                  _wspec((64, 128)), _wspec((1, 128))],
        out_specs=[_rowspec(_BLK_E, 64), _rowspec(_BLK_E, 128)],
        out_shape=[jax.ShapeDtypeStruct((_EP, 64), _F32),
                   jax.ShapeDtypeStruct((_EP, 128), _F32)],
    )(hs, hr, e0,
      we1p, be1.reshape(1, 64), we2, be2.reshape(1, 64), we3, be3.reshape(1, 64),
      wva, wvb, bv1.reshape(1, 64), wv2, bv2.reshape(1, 64), wv3p, bv3p)


def _edge_step2_force(hs, hr, e1, fe, mlp1):
    (we1, be1), (we2, be2), (we3, be3) = fe
    (wm1, bm1), (wm2, bm2), (wm3, bm3) = mlp1
    we1p = jnp.pad(we1, ((0, 64), (0, 0)))
    wm3p = jnp.pad(wm3, ((0, 0), (0, 125)))
    bm3p = jnp.pad(bm3.reshape(1, 3), ((0, 0), (0, 125)))

    def body(hs_ref, hr_ref, e1_ref,
             we1_r, be1_r, we2_r, be2_r, we3_r, be3_r,
             wm1_r, bm1_r, wm2_r, bm2_r, wm3_r, bm3_r,
             fij_o):
        prod = hs_ref[...] * hr_ref[...]
        t = _sp(_dot(prod, we1_r[...]) + be1_r[...])
        t = _sp(_dot(t, we2_r[...]) + be2_r[...])
        e2 = _dot(t, we3_r[...]) + be3_r[...] + e1_ref[...]
        f = _sp(_dot(e2, wm1_r[...]) + bm1_r[...])
        f = _sp(_dot(f, wm2_r[...]) + bm2_r[...])
        fij_o[...] = _dot(f, wm3_r[...]) + bm3_r[...]

    return pl.pallas_call(
        body,
        grid=(_GRID_E,),
        in_specs=[_rowspec(_BLK_E, 128), _rowspec(_BLK_E, 128),
                  _rowspec(_BLK_E, 64)] +
                 [_wspec((128, 64)), _wspec((1, 64))] +
                 [_wspec((64, 64)), _wspec((1, 64))] * 4 +
                 [_wspec((64, 128)), _wspec((1, 128))],
        out_specs=_rowspec(_BLK_E, 128),
        out_shape=jax.ShapeDtypeStruct((_EP, 128), _F32),
    )(hs, hr, e1,
      we1p, be1.reshape(1, 64), we2, be2.reshape(1, 64), we3, be3.reshape(1, 64),
      wm1, bm1.reshape(1, 64), wm2, bm2.reshape(1, 64), wm3p, bm3p)


def _combine_h(h0, p0, p1):
    def body(a_ref, b_ref, c_ref, o_ref):
        o_ref[...] = a_ref[...] + b_ref[...] + c_ref[...]

    return pl.pallas_call(
        body,
        grid=(_GRID_N,),
        in_specs=[_rowspec(_BLK_N, 128)] * 3,
        out_specs=_rowspec(_BLK_N, 128),
        out_shape=jax.ShapeDtypeStruct((_NP, 128), _F32),
    )(h0, p0, p1)


def _final_node(pr, ps, nt_p, mlp2):
    (w1, b1), (w2, b2), (w3, b3) = mlp2
    w1p = jnp.pad(w1, ((0, 3), (0, 3)))
    b1p = jnp.pad(b1.reshape(1, 5), ((0, 0), (0, 3)))
    w2p = jnp.pad(w2, ((0, 3), (0, 3)))
    b2p = jnp.pad(b2.reshape(1, 5), ((0, 0), (0, 3)))
    w3p = jnp.pad(w3, ((0, 3), (0, 7)))
    b3p = jnp.pad(b3.reshape(1, 1), ((0, 0), (0, 7)))

    def body(pr_r, ps_r, nt_r,
             w1_r, b1_r, w2_r, b2_r, w3_r, b3_r,
             ai_o, g_o):
        ai_o[...] = pr_r[...] - ps_r[...]
        g = _sp(_dot(nt_r[...], w1_r[...]) + b1_r[...])
        g = _sp(_dot(g, w2_r[...]) + b2_r[...])
        g_o[...] = _sp(_dot(g, w3_r[...]) + b3_r[...])

    return pl.pallas_call(
        body,
        grid=(_GRID_N,),
        in_specs=[_rowspec(_BLK_N, 8)] * 2 + [_rowspec(_BLK_N, 8)] +
                 [_wspec((8, 8)), _wspec((1, 8))] * 3,
        out_specs=[_rowspec(_BLK_N, 8), _rowspec(_BLK_N, 8)],
        out_shape=[jax.ShapeDtypeStruct((_NP, 8), _F32),
                   jax.ShapeDtypeStruct((_NP, 8), _F32)],
    )(pr, ps, nt_p, w1p, b1p, w2p, b2p, w3p, b3p)


_MESH = plsc.VectorSubcoreMesh(core_axis_name="c", subcore_axis_name="s")


def _gather_two(h_pad, sidx, ridx):
    @functools.partial(
        pl.kernel,
        out_type=[jax.ShapeDtypeStruct((_R, _W, 128), _F32)] * 2,
        mesh=_MESH,
        scratch_types=[pltpu.VMEM((_CH, _W), jnp.int32),
                       pltpu.VMEM((_W, 128), _F32),
                       pltpu.VMEM_SHARED((_NP, 128), _F32)],
    )
    def k(h_hbm, s_hbm, r_hbm, hs_out, hr_out, idx_v, buf, hsh):
        cid = lax.axis_index("c")
        sid = lax.axis_index("s")
        nsl = _NP // 16
        pltpu.sync_copy(h_hbm.at[pl.ds(sid * nsl, nsl)],
                        hsh.at[pl.ds(sid * nsl, nsl)])
        plsc.subcore_barrier()
        base = (sid * 2 + cid) * _RPW

        def run(idx_hbm, out_hbm):
            def chunk(ci, carry):
                row0 = base + ci * _CH
                pltpu.sync_copy(idx_hbm.at[pl.ds(row0, _CH)], idx_v)
                for g in range(_CH):
                    pltpu.sync_copy(hsh.at[idx_v.at[g]], buf)
                    pltpu.sync_copy(buf, out_hbm.at[row0 + g])
                return carry

            lax.fori_loop(0, _NCH, chunk, 0)

        run(s_hbm, hs_out)
        run(r_hbm, hr_out)

    hs, hr = k(h_pad, sidx, ridx)
    return hs.reshape(_EP, 128), hr.reshape(_EP, 128)


def _scatter_add64(vals, ridx, zeros_n):
    @functools.partial(
        pl.kernel,
        out_type=[jax.ShapeDtypeStruct((_NP, 128), _F32)] * 2,
        mesh=_MESH,
        scratch_types=[pltpu.VMEM((_CH, _W), jnp.int32),
                       pltpu.VMEM((_SSUB, _W, 128), _F32),
                       pltpu.VMEM_SHARED((_NP, 128), _F32),
                       pltpu.SemaphoreType.DMA],
    )
    def k(v_hbm, i_hbm, z_hbm, p0_out, p1_out, idx_v, rows_v, acc, sem):
        cid = lax.axis_index("c")
        sid = lax.axis_index("s")
        nsl = _NP // 16
        pltpu.sync_copy(z_hbm.at[pl.ds(sid * nsl, nsl)],
                        acc.at[pl.ds(sid * nsl, nsl)])
        plsc.subcore_barrier()
        base = cid * (_R // 2) + sid * _RPW

        def chunk(ci, carry):
            row0 = base + ci * _CH
            pltpu.sync_copy(i_hbm.at[pl.ds(row0, _CH)], idx_v)
            for g in range(_CH // _SSUB):
                pltpu.sync_copy(v_hbm.at[pl.ds(row0 + g * _SSUB, _SSUB)],
                                rows_v)
                for j in range(_SSUB):
                    pltpu.sync_copy(rows_v.at[j],
                                    acc.at[idx_v.at[g * _SSUB + j]], add=True)
            return carry

        lax.fori_loop(0, _NCH, chunk, 0)
        plsc.subcore_barrier()

        @pl.when(cid == 0)
        def _():
            pltpu.sync_copy(acc.at[pl.ds(sid * nsl, nsl)],
                            p0_out.at[pl.ds(sid * nsl, nsl)])

        @pl.when(cid == 1)
        def _():
            pltpu.sync_copy(acc.at[pl.ds(sid * nsl, nsl)],
                            p1_out.at[pl.ds(sid * nsl, nsl)])

    return k(vals, ridx, zeros_n)


def _scatter_fij(vals, ridx, sidx, zeros_n):
    @functools.partial(
        pl.kernel,
        out_type=[jax.ShapeDtypeStruct((_NP, 128), _F32)] * 2,
        mesh=_MESH,
        scratch_types=[pltpu.VMEM((_CH, _W), jnp.int32),
                       pltpu.VMEM((_SSUB, _W, 128), _F32),
                       pltpu.VMEM_SHARED((_NP, 128), _F32),
                       pltpu.SemaphoreType.DMA],
    )
    def k(v_hbm, ri_hbm, si_hbm, z_hbm, pr_out, ps_out,
          idx_v, rows_v, acc, sem):
        cid = lax.axis_index("c")
        sid = lax.axis_index("s")
        nsl = _NP // 16
        pltpu.sync_copy(z_hbm.at[pl.ds(sid * nsl, nsl)],
                        acc.at[pl.ds(sid * nsl, nsl)])
        plsc.subcore_barrier()
        rpt = _R // 16
        base = sid * rpt

        def make_chunk(idx_hbm):
            def chunk(ci, carry):
                row0 = base + ci * _CH
                pltpu.sync_copy(idx_hbm.at[pl.ds(row0, _CH)], idx_v)
                for g in range(_CH // _SSUB):
                    pltpu.sync_copy(v_hbm.at[pl.ds(row0 + g * _SSUB, _SSUB)],
                                    rows_v)
                    for j in range(_SSUB):
                        pltpu.sync_copy(rows_v.at[j],
                                        acc.at[idx_v.at[g * _SSUB + j]],
                                        add=True)
                return carry
            return chunk

        @pl.when(cid == 0)
        def _():
            lax.fori_loop(0, rpt // _CH, make_chunk(ri_hbm), 0)

        @pl.when(cid == 1)
        def _():
            lax.fori_loop(0, rpt // _CH, make_chunk(si_hbm), 0)

        plsc.subcore_barrier()

        @pl.when(cid == 0)
        def _():
            pltpu.sync_copy(acc.at[pl.ds(sid * nsl, nsl)],
                            pr_out.at[pl.ds(sid * nsl, nsl)])

        @pl.when(cid == 1)
        def _():
            pltpu.sync_copy(acc.at[pl.ds(sid * nsl, nsl)],
                            ps_out.at[pl.ds(sid * nsl, nsl)])

    return k(vals, ridx, sidx, zeros_n)


def kernel(x, edge_attr, node_type, edge_index, fa, fb, fe, fv, mlp1, mlp2):
    s = edge_index[0]
    r = edge_index[1]
    epad = _EP - _E
    sidx = jnp.concatenate([s, jnp.full((epad,), _N, jnp.int32)]).reshape(_R, _W)
    ridx = jnp.concatenate([r, jnp.full((epad,), _N, jnp.int32)]).reshape(_R, _W)
    x_p = jnp.pad(x, ((0, _NP - _N), (0, 0)))
    ea_p = jnp.pad(edge_attr, ((0, epad), (0, 0)))
    nt_p = jnp.pad(node_type, ((0, _NP - _N), (0, 3)))
    zeros_n = jnp.zeros((_NP, 128), _F32)

    h0 = _embed_nodes(x_p, fa)
    e0 = _embed_edges(ea_p, fb)

    hs0, hr0 = _gather_two(h0, sidx, ridx)
    e1, upd = _edge_step1(hs0, hr0, e0, fe, fv)
    p0, p1 = _scatter_add64(upd.reshape(_R, _W, 128), ridx, zeros_n)
    h1 = _combine_h(h0, p0, p1)

    hs1, hr1 = _gather_two(h1, sidx, ridx)
    fij = _edge_step2_force(hs1, hr1, e1, fe, mlp1)

    pr, ps = _scatter_fij(fij.reshape(_R, _W, 128), ridx, sidx, zeros_n)
    ai_pad, gamma_pad = _final_node(pr[:, :8], ps[:, :8], nt_p, mlp2)
    return ai_pad[:_N, :3], gamma_pad[:_N, :1]
